# Initial kernel scaffold; baseline (speedup 1.0000x reference)
import functools

import jax
import jax.numpy as jnp
from jax import lax
from jax.experimental import pallas as pl
from jax.experimental.pallas import tpu as pltpu
from jax.experimental.pallas import tpu_sc as plsc

NUM_LABELS = 1000000
D_MODEL = 64
BATCH = 16384
HIST = 50

B = BATCH * HIST
NC, NS = 2, 16
NW = NC * NS
B_PER_W = B // NW
CHUNK = 128
N_CHUNKS = B_PER_W // CHUNK
NBUF = 4
N_OUTER = N_CHUNKS // NBUF


def _emb_kernel(idx_hbm, table_hbm, out_hbm, idx_v, rows, gsems, osems):
    wid = lax.axis_index("s") * NC + lax.axis_index("c")
    chunk0 = wid * N_CHUNKS

    pltpu.sync_copy(idx_hbm.at[pl.ds(chunk0, N_CHUNKS)], idx_v)

    def start_gather(j, b):
        pltpu.async_copy(table_hbm.at[idx_v.at[j]], rows[b], gsems[b])

    def wait_gather(b):
        pltpu.make_async_copy(table_hbm.at[idx_v.at[0]], rows[b], gsems[b]).wait()

    def start_out(j, b):
        dst = out_hbm.at[pl.ds((chunk0 + j) * CHUNK, CHUNK)]
        pltpu.async_copy(rows[b], dst, osems[b])

    def wait_out(b):
        dst = out_hbm.at[pl.ds(chunk0 * CHUNK, CHUNK)]
        pltpu.make_async_copy(rows[b], dst, osems[b]).wait()

    for b in range(NBUF):
        start_gather(b, b)

    def outer(g, carry):
        base = g * NBUF
        for b in range(NBUF):
            j = base + b
            wait_gather(b)
            start_out(j, b)
            wait_out(b)

            @pl.when(g + 1 < N_OUTER)
            def _():
                start_gather(j + NBUF, b)

        return carry

    lax.fori_loop(0, N_OUTER, outer, 0)


@jax.jit
def kernel(x, table):
    idx = x.reshape(B // CHUNK, CHUNK).astype(jnp.int32)
    mesh = plsc.VectorSubcoreMesh(core_axis_name="c", subcore_axis_name="s")
    out = pl.kernel(
        _emb_kernel,
        out_type=jax.ShapeDtypeStruct((B, D_MODEL), jnp.float32),
        mesh=mesh,
        scratch_types=[
            pltpu.VMEM((N_CHUNKS, CHUNK), jnp.int32),
            [pltpu.VMEM((CHUNK, D_MODEL), jnp.float32) for _ in range(NBUF)],
            [pltpu.SemaphoreType.DMA for _ in range(NBUF)],
            [pltpu.SemaphoreType.DMA for _ in range(NBUF)],
        ],
    )(idx, table)
    return out.reshape(BATCH, HIST, D_MODEL)


# speedup vs baseline: 1.8761x; 1.8761x over previous
import functools

import jax
import jax.numpy as jnp
from jax import lax
from jax.experimental import pallas as pl
from jax.experimental.pallas import tpu as pltpu
from jax.experimental.pallas import tpu_sc as plsc

NUM_LABELS = 1000000
D_MODEL = 64
BATCH = 16384
HIST = 50

B = BATCH * HIST
NC, NS = 2, 16
NW = NC * NS
B_PER_W = B // NW
CHUNK = 128
N_CHUNKS = B_PER_W // CHUNK
NBUF = 4
N_OUTER = N_CHUNKS // NBUF


def _emb_kernel(idx_hbm, table_hbm, out_hbm, idx_v, rows, gsems, osems):
    wid = lax.axis_index("s") * NC + lax.axis_index("c")
    chunk0 = wid * N_CHUNKS

    pltpu.sync_copy(idx_hbm.at[pl.ds(chunk0, N_CHUNKS)], idx_v)

    def start_gather(j, b):
        pltpu.async_copy(table_hbm.at[idx_v.at[j]], rows[b], gsems[b])

    def wait_gather(b):
        pltpu.make_async_copy(table_hbm.at[idx_v.at[0]], rows[b], gsems[b]).wait()

    def start_out(j, b):
        dst = out_hbm.at[pl.ds((chunk0 + j) * CHUNK, CHUNK)]
        pltpu.async_copy(rows[b], dst, osems[b])

    def wait_out(b):
        dst = out_hbm.at[pl.ds(chunk0 * CHUNK, CHUNK)]
        pltpu.make_async_copy(rows[b], dst, osems[b]).wait()

    for b in range(NBUF):
        start_gather(b, b)

    def outer(g, carry):
        base = g * NBUF
        for b in range(NBUF):
            j = base + b
            wait_gather(b)
            start_out(j, b)
            wait_out(b)

            @pl.when(g + 1 < N_OUTER)
            def _():
                start_gather(j + NBUF, b)

        return carry

    lax.fori_loop(0, N_OUTER, outer, 0)


@jax.jit
def kernel(x, table):
    idx = x.reshape(B // CHUNK, CHUNK).astype(jnp.int32)
    mesh = plsc.VectorSubcoreMesh(core_axis_name="c", subcore_axis_name="s")
    out = pl.kernel(
        _emb_kernel,
        out_type=jax.ShapeDtypeStruct((B, D_MODEL), jnp.float32),
        mesh=mesh,
        scratch_types=[
            pltpu.VMEM((N_CHUNKS, CHUNK), jnp.int32),
            [pltpu.VMEM((CHUNK, D_MODEL), jnp.float32) for _ in range(NBUF)],
            [pltpu.SemaphoreType.DMA for _ in range(NBUF)],
            [pltpu.SemaphoreType.DMA for _ in range(NBUF)],
        ],
        compiler_params=pltpu.CompilerParams(use_tc_tiling_on_sc=False),
    )(idx, table)
    return out.reshape(BATCH, HIST, D_MODEL)


# speedup vs baseline: 1.8881x; 1.0064x over previous
import functools

import jax
import jax.numpy as jnp
from jax import lax
from jax.experimental import pallas as pl
from jax.experimental.pallas import tpu as pltpu
from jax.experimental.pallas import tpu_sc as plsc

NUM_LABELS = 1000000
D_MODEL = 64
BATCH = 16384
HIST = 50

B = BATCH * HIST
NC, NS = 2, 16
NW = NC * NS
B_PER_W = B // NW
CHUNK = 128
N_CHUNKS = B_PER_W // CHUNK
NRING = 8
LOOK = 4
N_OUTER = N_CHUNKS // NRING


def _emb_kernel(idx_hbm, table_hbm, out_hbm, idx_v, rows, gsems, osems):
    wid = lax.axis_index("s") * NC + lax.axis_index("c")
    chunk0 = wid * N_CHUNKS

    pltpu.sync_copy(idx_hbm.at[pl.ds(chunk0, N_CHUNKS)], idx_v)

    def start_gather(j, b):
        pltpu.async_copy(table_hbm.at[idx_v.at[j]], rows[b], gsems[b])

    def wait_gather(b):
        pltpu.make_async_copy(table_hbm.at[idx_v.at[0]], rows[b], gsems[b]).wait()

    def start_out(j, b):
        dst = out_hbm.at[pl.ds((chunk0 + j) * CHUNK, CHUNK)]
        pltpu.async_copy(rows[b], dst, osems[b])

    def wait_out(b):
        dst = out_hbm.at[pl.ds(chunk0 * CHUNK, CHUNK)]
        pltpu.make_async_copy(rows[b], dst, osems[b]).wait()

    for b in range(LOOK):
        start_gather(b, b)

    def outer(g, carry):
        base = g * NRING
        for b in range(NRING):
            j = base + b
            wait_gather(b)
            start_out(j, b)

            t = j + LOOK
            bt = (b + LOOK) % NRING
            if b < LOOK:
                @pl.when(g >= 1)
                def _():
                    wait_out(bt)

                start_gather(t, bt)
            else:
                @pl.when(g < N_OUTER - 1)
                def _():
                    wait_out(bt)
                    start_gather(t, bt)

        return carry

    lax.fori_loop(0, N_OUTER, outer, 0)

    for b in range(NRING):
        wait_out(b)


@jax.jit
def kernel(x, table):
    idx = x.reshape(B // CHUNK, CHUNK).astype(jnp.int32)
    mesh = plsc.VectorSubcoreMesh(core_axis_name="c", subcore_axis_name="s")
    out = pl.kernel(
        _emb_kernel,
        out_type=jax.ShapeDtypeStruct((B, D_MODEL), jnp.float32),
        mesh=mesh,
        scratch_types=[
            pltpu.VMEM((N_CHUNKS, CHUNK), jnp.int32),
            [pltpu.VMEM((CHUNK, D_MODEL), jnp.float32) for _ in range(NRING)],
            [pltpu.SemaphoreType.DMA for _ in range(NRING)],
            [pltpu.SemaphoreType.DMA for _ in range(NRING)],
        ],
        compiler_params=pltpu.CompilerParams(use_tc_tiling_on_sc=False),
    )(idx, table)
    return out.reshape(BATCH, HIST, D_MODEL)
